# Initial kernel scaffold; baseline (speedup 1.0000x reference)
import jax
import jax.numpy as jnp
from jax.experimental import pallas as pl


def kernel(x, emb_table, freq_weights, freq_proj_w, freq_proj_b):
    raise NotImplementedError("write your pallas kernel here")


# speedup vs baseline: 1.0147x; 1.0147x over previous
import functools

import jax
import jax.numpy as jnp
from jax import lax
from jax.experimental import pallas as pl
from jax.experimental.pallas import tpu as pltpu
from jax.experimental.pallas import tpu_sc as plsc


def kernel(x, emb_table, freq_weights, freq_proj_w, freq_proj_b):
    B, L = x.shape
    V, D = emb_table.shape
    N = B * L

    idx = x.reshape(N).astype(jnp.int32)
    wb = jnp.concatenate(
        [0.1 * freq_proj_w[:, 0], 0.1 * freq_proj_b]
    ).astype(jnp.float32)

    info = plsc.get_sparse_core_info()
    NC, NS = info.num_cores, info.num_subcores
    NW = NC * NS
    per_w = N // NW
    CH = 1600
    n_ch = per_w // CH

    mesh = plsc.VectorSubcoreMesh(core_axis_name="c", subcore_axis_name="s")

    @functools.partial(
        pl.kernel,
        mesh=mesh,
        out_type=jax.ShapeDtypeStruct((N, D), jnp.float32),
        compiler_params=pltpu.CompilerParams(use_tc_tiling_on_sc=False),
        scratch_types=[
            pltpu.VMEM((CH,), jnp.int32),
            pltpu.VMEM((CH,), jnp.float32),
            pltpu.VMEM((CH, D), jnp.float32),
            pltpu.VMEM((2 * D,), jnp.float32),
            pltpu.SemaphoreType.DMA,
            pltpu.SemaphoreType.DMA,
        ],
    )
    def sc_embed(idx_hbm, tab_hbm, fw_hbm, wb_hbm, out_hbm,
                 idx_v, fv_v, rows_v, wb_v, sem_r, sem_f):
        wid = lax.axis_index("s") * NC + lax.axis_index("c")
        pltpu.sync_copy(wb_hbm, wb_v)
        w0 = wb_v[pl.ds(0, 16)]
        w1 = wb_v[pl.ds(16, 16)]
        b0 = wb_v[pl.ds(32, 16)]
        b1 = wb_v[pl.ds(48, 16)]

        def chunk_body(c, carry):
            base = wid * per_w + c * CH
            pltpu.sync_copy(idx_hbm.at[pl.ds(base, CH)], idx_v)
            gr = pltpu.async_copy(tab_hbm.at[idx_v], rows_v, sem_r)
            gf = pltpu.async_copy(fw_hbm.at[idx_v], fv_v, sem_f)
            gr.wait()
            gf.wait()

            def blk_body(j, carry2):
                fvb = fv_v[pl.ds(j * 16, 16)]
                for k in range(16):
                    i = j * 16 + k
                    s = jnp.take_along_axis(
                        fvb, jnp.full((16,), k, jnp.int32), axis=0)
                    rows_v[i, pl.ds(0, 16)] = rows_v[i, pl.ds(0, 16)] + s * w0 + b0
                    rows_v[i, pl.ds(16, 16)] = rows_v[i, pl.ds(16, 16)] + s * w1 + b1
                return carry2

            lax.fori_loop(0, CH // 16, blk_body, 0)
            pltpu.sync_copy(rows_v, out_hbm.at[pl.ds(base, CH)])
            return carry

        lax.fori_loop(0, n_ch, chunk_body, 0)

    out = sc_embed(idx, emb_table, freq_weights, wb)
    return out.reshape(B, L, D)
